# Initial kernel scaffold; baseline (speedup 1.0000x reference)
import jax
import jax.numpy as jnp
from jax.experimental import pallas as pl


def kernel(x_user, x_movie, edge_src, edge_dst, label_src, label_dst, W1l_r, b1_r, W1r_r, W1l_rb, b1_rb, W1r_rb, W2l_r, b2_r, W2r_r, W2l_rb, b2_rb, W2r_rb, Wc1, bc1, Wc2, bc2, Wc3, bc3):
    raise NotImplementedError("write your pallas kernel here")


# speedup vs baseline: 5.6528x; 5.6528x over previous
import functools

import jax
import jax.numpy as jnp
from jax import lax
from jax.experimental import pallas as pl
from jax.experimental.pallas import tpu as pltpu
from jax.experimental.pallas import tpu_sc as plsc

NU = 50000
NM = 10000
E = 600000
L = 100000
H = 128

NC = 2
NS = 16
NW = NC * NS

W32 = 32
NU_A = 50176
NM_A = 10240
ZU = NU_A // NS
ZM = NM_A // NS

PT = 152
KB_M = 8
KB_U = 4
E_PAD = NW * PT * 128
EC = E_PAD // 128
PC = EC // NC

PTC = 25
KBC = 5
NBLKC = PTC // KBC
L_PAD = NW * PTC * 128
LC = L_PAD // 128

BR = 512

_f32 = jnp.float32
_i32 = jnp.int32


def _seg_dir_sc(gidx2, sidx2, tab, zrows, acc_rows, kb):
  nblk = PT // kb
  zr = acc_rows // NS

  def body(g_hbm, s_hbm, tab_hbm, zero_hbm, acc_out,
           idx_g, idx_s, rows, acc, gsem):
    c = lax.axis_index("c")
    s = lax.axis_index("s")
    pltpu.sync_copy(zero_hbm.at[pl.ds(0, zr)], acc.at[pl.ds(s * zr, zr)])
    plsc.subcore_barrier()

    start = c * PC + s * PT

    def blk(b, carry):
      base = start + b * kb
      pltpu.sync_copy(g_hbm.at[pl.ds(base, kb)], idx_g)
      pltpu.sync_copy(s_hbm.at[pl.ds(base, kb)], idx_s)
      descs = []
      for j in range(kb):
        descs.append(pltpu.async_copy(
            tab_hbm.at[idx_g.at[j]], rows.at[pl.ds(j * 128, 128)], gsem))
      for d in descs:
        d.wait()
      for j in range(kb):
        pltpu.sync_copy(rows.at[pl.ds(j * 128, 128)],
                        acc.at[idx_s.at[j]], add=True)
      return carry

    lax.fori_loop(0, nblk, blk, 0)
    plsc.subcore_barrier()
    pltpu.sync_copy(acc.at[pl.ds(s * zr, zr)],
                    acc_out.at[c, pl.ds(s * zr, zr)])

  mesh = plsc.VectorSubcoreMesh(core_axis_name="c", subcore_axis_name="s")
  f = pl.kernel(
      body,
      out_type=[jax.ShapeDtypeStruct((NC, acc_rows, W32), _f32)],
      mesh=mesh,
      scratch_types=[
          pltpu.VMEM((kb, 128), _i32),
          pltpu.VMEM((kb, 128), _i32),
          pltpu.VMEM((kb * 128, W32), _f32),
          pltpu.VMEM_SHARED((acc_rows, W32), _f32),
          pltpu.SemaphoreType.DMA,
      ],
      compiler_params=pltpu.CompilerParams(use_tc_tiling_on_sc=False),
  )
  (out,) = f(gidx2, sidx2, tab, zrows)
  return out


def _gather_sc(lsrc, ldst, tab_u, tab_m):
  def body(ls_hbm, ld_hbm, tu_hbm, tm_hbm, gu_out, gm_out,
           idx_s, idx_d, rows_u, rows_m, gsem):
    c = lax.axis_index("c")
    s = lax.axis_index("s")
    start = (c * NS + s) * PTC

    def blk(b, carry):
      base = (start + b * KBC) * 128
      pltpu.sync_copy(ls_hbm.at[pl.ds(base, KBC * 128)], idx_s)
      pltpu.sync_copy(ld_hbm.at[pl.ds(base, KBC * 128)], idx_d)
      descs = []
      for j in range(KBC):
        descs.append(pltpu.async_copy(
            tu_hbm.at[idx_s.at[pl.ds(j * 128, 128)]],
            rows_u.at[pl.ds(j * 128, 128)], gsem))
        descs.append(pltpu.async_copy(
            tm_hbm.at[idx_d.at[pl.ds(j * 128, 128)]],
            rows_m.at[pl.ds(j * 128, 128)], gsem))
      for d in descs:
        d.wait()
      pltpu.sync_copy(rows_u, gu_out.at[pl.ds(base, KBC * 128)])
      pltpu.sync_copy(rows_m, gm_out.at[pl.ds(base, KBC * 128)])
      return carry

    lax.fori_loop(0, NBLKC, blk, 0)

  mesh = plsc.VectorSubcoreMesh(core_axis_name="c", subcore_axis_name="s")
  f = pl.kernel(
      body,
      out_type=[
          jax.ShapeDtypeStruct((L_PAD, 64), _f32),
          jax.ShapeDtypeStruct((L_PAD, 64), _f32),
      ],
      mesh=mesh,
      scratch_types=[
          pltpu.VMEM((KBC * 128,), _i32),
          pltpu.VMEM((KBC * 128,), _i32),
          pltpu.VMEM((KBC * 128, 64), _f32),
          pltpu.VMEM((KBC * 128, 64), _f32),
          pltpu.SemaphoreType.DMA,
      ],
      compiler_params=pltpu.CompilerParams(use_tc_tiling_on_sc=False),
  )
  return f(lsrc, ldst, tab_u, tab_m)


def _agg_table_tc(s_parts, feat):
  n = s_parts.shape[1]

  def body(sp_ref, out_ref):
    S = sp_ref[0] + sp_ref[1]
    m = jnp.maximum(S[:, feat], 1.0)
    col = lax.broadcasted_iota(_i32, (BR, W32), 1)
    out_ref[...] = jnp.where(col < feat, S / m[:, None], 0.0)

  return pl.pallas_call(
      body,
      grid=(n // BR,),
      in_specs=[pl.BlockSpec((NC, BR, W32), lambda i: (0, i, 0))],
      out_specs=pl.BlockSpec((BR, W32), lambda i: (i, 0)),
      out_shape=jax.ShapeDtypeStruct((n, W32), _f32),
  )(s_parts)


def _a_table_tc(q_parts, s_parts, x_tab, wq, ws, wx, vb, cb, feat):
  n = x_tab.shape[0]

  def body(q_ref, s_ref, x_ref, wq_ref, ws_ref, wx_ref, vb_ref, cb_ref,
           out_ref):
    Q = q_ref[0] + q_ref[1]
    S = s_ref[0] + s_ref[1]
    deg = S[:, feat]
    m = jnp.maximum(deg, 1.0)
    br = (jnp.dot(Q, wq_ref[...], preferred_element_type=_f32)
          + jnp.dot(S, ws_ref[...], preferred_element_type=_f32)
          + deg[:, None] * vb_ref[...])
    out_ref[...] = (br / m[:, None]
                    + jnp.dot(x_ref[...], wx_ref[...],
                              preferred_element_type=_f32)
                    + cb_ref[...])

  full = lambda i: (0, 0)
  return pl.pallas_call(
      body,
      grid=(n // BR,),
      in_specs=[
          pl.BlockSpec((NC, BR, W32), lambda i: (0, i, 0)),
          pl.BlockSpec((NC, BR, W32), lambda i: (0, i, 0)),
          pl.BlockSpec((BR, W32), lambda i: (i, 0)),
          pl.BlockSpec((W32, 64), full),
          pl.BlockSpec((W32, 64), full),
          pl.BlockSpec((W32, 64), full),
          pl.BlockSpec((1, 64), full),
          pl.BlockSpec((1, 64), full),
      ],
      out_specs=pl.BlockSpec((BR, 64), lambda i: (i, 0)),
      out_shape=jax.ShapeDtypeStruct((n, 64), _f32),
  )(q_parts, s_parts, x_tab, wq, ws, wx, vb, cb)


def _classifier_tc(gu, gm, b1, w2, b2, w3row, b3):
  def body(gu_ref, gm_ref, b1_ref, w2_ref, b2_ref, w3_ref, b3_ref, out_ref):
    h = jnp.maximum(gu_ref[...] + gm_ref[...] + b1_ref[...], 0.0)
    h2 = jnp.maximum(
        jnp.dot(h, w2_ref[...], preferred_element_type=_f32) + b2_ref[...],
        0.0)
    z = jnp.sum(h2 * w3_ref[...], axis=1) + b3_ref[0, 0]
    out_ref[...] = (5.0 / (1.0 + jnp.exp(-z)))[:, None]

  full = lambda i: (0, 0)
  return pl.pallas_call(
      body,
      grid=(L_PAD // BR,),
      in_specs=[
          pl.BlockSpec((BR, 64), lambda i: (i, 0)),
          pl.BlockSpec((BR, 64), lambda i: (i, 0)),
          pl.BlockSpec((1, 64), full),
          pl.BlockSpec((64, 16), full),
          pl.BlockSpec((1, 16), full),
          pl.BlockSpec((1, 16), full),
          pl.BlockSpec((1, 1), full),
      ],
      out_specs=pl.BlockSpec((BR, 1), lambda i: (i, 0)),
      out_shape=jax.ShapeDtypeStruct((L_PAD, 1), _f32),
  )(gu, gm, b1, w2, b2, w3row, b3)


def kernel(x_user, x_movie, edge_src, edge_dst, label_src, label_dst,
           W1l_r, b1_r, W1r_r, W1l_rb, b1_rb, W1r_rb,
           W2l_r, b2_r, W2r_r, W2l_rb, b2_rb, W2r_rb,
           Wc1, bc1, Wc2, bc2, Wc3, bc3):
  es = edge_src.astype(_i32)
  ed = edge_dst.astype(_i32)
  src2 = jnp.concatenate(
      [es, jnp.full((E_PAD - E,), NU, _i32)]).reshape(EC, 128)
  dst2 = jnp.concatenate(
      [ed, jnp.full((E_PAD - E,), NM, _i32)]).reshape(EC, 128)
  ls1 = jnp.concatenate(
      [label_src.astype(_i32), jnp.zeros((L_PAD - L,), _i32)])
  ld1 = jnp.concatenate(
      [label_dst.astype(_i32), jnp.zeros((L_PAD - L,), _i32)])

  xu_ext = jnp.zeros((NU_A, W32), _f32)
  xu_ext = xu_ext.at[:NU, :24].set(x_user).at[:NU, 24].set(1.0)
  xm_ext = jnp.zeros((NM_A, W32), _f32)
  xm_ext = xm_ext.at[:NM, :18].set(x_movie).at[:NM, 18].set(1.0)
  zrows = jnp.zeros((ZU, W32), _f32)

  Wc1u, Wc1m = Wc1[:H], Wc1[H:]
  V1 = W2l_rb @ Wc1u
  W2ru = W2r_rb @ Wc1u
  M1 = W1l_r @ V1
  M23 = W1r_r @ V1 + W1l_rb @ W2ru
  v1 = (b1_r @ V1)[None, :]
  V4 = W1r_rb @ W2ru
  c_u = ((b1_rb @ W2r_rb + b2_rb) @ Wc1u)[None, :]

  V2m = W2l_r @ Wc1m
  W2rm = W2r_r @ Wc1m
  N1 = W1l_rb @ V2m
  N23 = W1r_rb @ V2m + W1l_r @ W2rm
  v2 = (b1_rb @ V2m)[None, :]
  N4 = W1r_r @ W2rm
  c_m = ((b1_r @ W2r_r + b2_r) @ Wc1m)[None, :]

  pad32 = lambda w: jnp.zeros((W32, 64), _f32).at[:w.shape[0]].set(w)
  M1p, M23p, V4p = pad32(M1), pad32(M23), pad32(V4)
  N1p, N23p, N4p = pad32(N1), pad32(N23), pad32(N4)

  Sm_parts = _seg_dir_sc(src2, dst2, xu_ext, zrows, NM_A, KB_M)
  Su_parts = _seg_dir_sc(dst2, src2, xm_ext, zrows, NU_A, KB_U)

  aggU = _agg_table_tc(Su_parts, 18)
  aggM = _agg_table_tc(Sm_parts, 24)

  Pm_parts = _seg_dir_sc(src2, dst2, aggU, zrows, NM_A, KB_M)
  Qu_parts = _seg_dir_sc(dst2, src2, aggM, zrows, NU_A, KB_U)

  A_u = _a_table_tc(Qu_parts, Su_parts, xu_ext, M1p, M23p, V4p, v1, c_u, 18)
  A_m = _a_table_tc(Pm_parts, Sm_parts, xm_ext, N1p, N23p, N4p, v2, c_m, 24)

  Gu, Gm = _gather_sc(ls1, ld1, A_u, A_m)

  out2 = _classifier_tc(Gu, Gm, bc1[None, :], Wc2, bc2[None, :],
                        Wc3[:, 0][None, :], bc3[None, :])
  return out2.reshape(L_PAD)[:L]


# speedup vs baseline: 6.8138x; 1.2054x over previous
import functools

import jax
import jax.numpy as jnp
from jax import lax
from jax.experimental import pallas as pl
from jax.experimental.pallas import tpu as pltpu
from jax.experimental.pallas import tpu_sc as plsc

NU = 50000
NM = 10000
E = 600000
L = 100000
H = 128

NC = 2
NS = 16
NW = NC * NS

W32 = 32
NU_A = 50176
NM_A = 10240
ZU = NU_A // NS
ZM = NM_A // NS

TPT = 304
SB = 8
HB = 4
NSB = TPT // SB
EC = NS * TPT
E_PAD = EC * 128

PTC = 25
KBC = 5
NBLKC = PTC // KBC
L_PAD = NW * PTC * 128
LC = L_PAD // 128

BR = 512

_f32 = jnp.float32
_i32 = jnp.int32


def _seg_sum_sc(src2, dst2, tab_u, tab_m, zrows):
  def body(src_hbm, dst_hbm, tabu_hbm, tabm_hbm, zero_hbm,
           accm_out, accu_out,
           idx_g, idx_s, rows, acc, gsem, ssem):
    c = lax.axis_index("c")
    s = lax.axis_index("s")

    def run(g_hbm, s_hbm, tab_hbm, nrows, out_hbm):
      zr = nrows // NS
      pltpu.sync_copy(zero_hbm.at[pl.ds(0, zr)], acc.at[pl.ds(s * zr, zr)])
      plsc.subcore_barrier()
      start = s * TPT

      def blk(b, carry):
        base = start + b * SB
        pltpu.sync_copy(g_hbm.at[pl.ds(base, SB)], idx_g)
        pltpu.sync_copy(s_hbm.at[pl.ds(base, SB)], idx_s)
        for h in range(SB // HB):
          gd = [pltpu.async_copy(
                    tab_hbm.at[idx_g.at[h * HB + j]],
                    rows.at[pl.ds(j * 128, 128)], gsem)
                for j in range(HB)]
          for d in gd:
            d.wait()
          sd = [pltpu.async_copy(
                    rows.at[pl.ds(j * 128, 128)],
                    acc.at[idx_s.at[h * HB + j]], ssem, add=True)
                for j in range(HB)]
          for d in sd:
            d.wait()
        return carry

      lax.fori_loop(0, NSB, blk, 0)
      plsc.subcore_barrier()
      pltpu.sync_copy(acc.at[pl.ds(s * zr, zr)], out_hbm.at[pl.ds(s * zr, zr)])

    @pl.when(c == 0)
    def _():
      run(src_hbm, dst_hbm, tabu_hbm, NM_A, accm_out)

    @pl.when(c == 1)
    def _():
      run(dst_hbm, src_hbm, tabm_hbm, NU_A, accu_out)

  mesh = plsc.VectorSubcoreMesh(core_axis_name="c", subcore_axis_name="s")
  f = pl.kernel(
      body,
      out_type=[
          jax.ShapeDtypeStruct((NM_A, W32), _f32),
          jax.ShapeDtypeStruct((NU_A, W32), _f32),
      ],
      mesh=mesh,
      scratch_types=[
          pltpu.VMEM((SB, 128), _i32),
          pltpu.VMEM((SB, 128), _i32),
          pltpu.VMEM((HB * 128, W32), _f32),
          pltpu.VMEM_SHARED((NU_A, W32), _f32),
          pltpu.SemaphoreType.DMA,
          pltpu.SemaphoreType.DMA,
      ],
      compiler_params=pltpu.CompilerParams(use_tc_tiling_on_sc=False),
  )
  return f(src2, dst2, tab_u, tab_m, zrows)


def _gather_sc(lsrc, ldst, tab_u, tab_m):
  def body(ls_hbm, ld_hbm, tu_hbm, tm_hbm, gu_out, gm_out,
           idx_s, idx_d, rows_u, rows_m, gsem):
    c = lax.axis_index("c")
    s = lax.axis_index("s")
    start = (c * NS + s) * PTC

    def blk(b, carry):
      base = (start + b * KBC) * 128
      pltpu.sync_copy(ls_hbm.at[pl.ds(base, KBC * 128)], idx_s)
      pltpu.sync_copy(ld_hbm.at[pl.ds(base, KBC * 128)], idx_d)
      descs = []
      for j in range(KBC):
        descs.append(pltpu.async_copy(
            tu_hbm.at[idx_s.at[pl.ds(j * 128, 128)]],
            rows_u.at[pl.ds(j * 128, 128)], gsem))
        descs.append(pltpu.async_copy(
            tm_hbm.at[idx_d.at[pl.ds(j * 128, 128)]],
            rows_m.at[pl.ds(j * 128, 128)], gsem))
      for d in descs:
        d.wait()
      pltpu.sync_copy(rows_u, gu_out.at[pl.ds(base, KBC * 128)])
      pltpu.sync_copy(rows_m, gm_out.at[pl.ds(base, KBC * 128)])
      return carry

    lax.fori_loop(0, NBLKC, blk, 0)

  mesh = plsc.VectorSubcoreMesh(core_axis_name="c", subcore_axis_name="s")
  f = pl.kernel(
      body,
      out_type=[
          jax.ShapeDtypeStruct((L_PAD, 64), _f32),
          jax.ShapeDtypeStruct((L_PAD, 64), _f32),
      ],
      mesh=mesh,
      scratch_types=[
          pltpu.VMEM((KBC * 128,), _i32),
          pltpu.VMEM((KBC * 128,), _i32),
          pltpu.VMEM((KBC * 128, 64), _f32),
          pltpu.VMEM((KBC * 128, 64), _f32),
          pltpu.SemaphoreType.DMA,
      ],
      compiler_params=pltpu.CompilerParams(use_tc_tiling_on_sc=False),
  )
  return f(lsrc, ldst, tab_u, tab_m)


def _agg_table_tc(s_tab, feat):
  n = s_tab.shape[0]

  def body(sp_ref, out_ref):
    S = sp_ref[...]
    m = jnp.maximum(S[:, feat], 1.0)
    col = lax.broadcasted_iota(_i32, (BR, W32), 1)
    out_ref[...] = jnp.where(col < feat, S / m[:, None], 0.0)

  return pl.pallas_call(
      body,
      grid=(n // BR,),
      in_specs=[pl.BlockSpec((BR, W32), lambda i: (i, 0))],
      out_specs=pl.BlockSpec((BR, W32), lambda i: (i, 0)),
      out_shape=jax.ShapeDtypeStruct((n, W32), _f32),
  )(s_tab)


def _a_table_tc(q_tab, s_tab, x_tab, wq, ws, wx, vb, cb, feat):
  n = x_tab.shape[0]

  def body(q_ref, s_ref, x_ref, wq_ref, ws_ref, wx_ref, vb_ref, cb_ref,
           out_ref):
    Q = q_ref[...]
    S = s_ref[...]
    deg = S[:, feat]
    m = jnp.maximum(deg, 1.0)
    br = (jnp.dot(Q, wq_ref[...], preferred_element_type=_f32)
          + jnp.dot(S, ws_ref[...], preferred_element_type=_f32)
          + deg[:, None] * vb_ref[...])
    out_ref[...] = (br / m[:, None]
                    + jnp.dot(x_ref[...], wx_ref[...],
                              preferred_element_type=_f32)
                    + cb_ref[...])

  full = lambda i: (0, 0)
  return pl.pallas_call(
      body,
      grid=(n // BR,),
      in_specs=[
          pl.BlockSpec((BR, W32), lambda i: (i, 0)),
          pl.BlockSpec((BR, W32), lambda i: (i, 0)),
          pl.BlockSpec((BR, W32), lambda i: (i, 0)),
          pl.BlockSpec((W32, 64), full),
          pl.BlockSpec((W32, 64), full),
          pl.BlockSpec((W32, 64), full),
          pl.BlockSpec((1, 64), full),
          pl.BlockSpec((1, 64), full),
      ],
      out_specs=pl.BlockSpec((BR, 64), lambda i: (i, 0)),
      out_shape=jax.ShapeDtypeStruct((n, 64), _f32),
  )(q_tab, s_tab, x_tab, wq, ws, wx, vb, cb)


def _classifier_tc(gu, gm, b1, w2, b2, w3row, b3):
  def body(gu_ref, gm_ref, b1_ref, w2_ref, b2_ref, w3_ref, b3_ref, out_ref):
    h = jnp.maximum(gu_ref[...] + gm_ref[...] + b1_ref[...], 0.0)
    h2 = jnp.maximum(
        jnp.dot(h, w2_ref[...], preferred_element_type=_f32) + b2_ref[...],
        0.0)
    z = jnp.sum(h2 * w3_ref[...], axis=1) + b3_ref[0, 0]
    out_ref[...] = (5.0 / (1.0 + jnp.exp(-z)))[:, None]

  full = lambda i: (0, 0)
  return pl.pallas_call(
      body,
      grid=(L_PAD // BR,),
      in_specs=[
          pl.BlockSpec((BR, 64), lambda i: (i, 0)),
          pl.BlockSpec((BR, 64), lambda i: (i, 0)),
          pl.BlockSpec((1, 64), full),
          pl.BlockSpec((64, 16), full),
          pl.BlockSpec((1, 16), full),
          pl.BlockSpec((1, 16), full),
          pl.BlockSpec((1, 1), full),
      ],
      out_specs=pl.BlockSpec((BR, 1), lambda i: (i, 0)),
      out_shape=jax.ShapeDtypeStruct((L_PAD, 1), _f32),
  )(gu, gm, b1, w2, b2, w3row, b3)


def kernel(x_user, x_movie, edge_src, edge_dst, label_src, label_dst,
           W1l_r, b1_r, W1r_r, W1l_rb, b1_rb, W1r_rb,
           W2l_r, b2_r, W2r_r, W2l_rb, b2_rb, W2r_rb,
           Wc1, bc1, Wc2, bc2, Wc3, bc3):
  es = edge_src.astype(_i32)
  ed = edge_dst.astype(_i32)
  src2 = jnp.concatenate(
      [es, jnp.full((E_PAD - E,), NU, _i32)]).reshape(EC, 128)
  dst2 = jnp.concatenate(
      [ed, jnp.full((E_PAD - E,), NM, _i32)]).reshape(EC, 128)
  ls1 = jnp.concatenate(
      [label_src.astype(_i32), jnp.zeros((L_PAD - L,), _i32)])
  ld1 = jnp.concatenate(
      [label_dst.astype(_i32), jnp.zeros((L_PAD - L,), _i32)])

  xu_ext = jnp.zeros((NU_A, W32), _f32)
  xu_ext = xu_ext.at[:NU, :24].set(x_user).at[:NU, 24].set(1.0)
  xm_ext = jnp.zeros((NM_A, W32), _f32)
  xm_ext = xm_ext.at[:NM, :18].set(x_movie).at[:NM, 18].set(1.0)
  zrows = jnp.zeros((ZU, W32), _f32)

  Wc1u, Wc1m = Wc1[:H], Wc1[H:]
  V1 = W2l_rb @ Wc1u
  W2ru = W2r_rb @ Wc1u
  M1 = W1l_r @ V1
  M23 = W1r_r @ V1 + W1l_rb @ W2ru
  v1 = (b1_r @ V1)[None, :]
  V4 = W1r_rb @ W2ru
  c_u = ((b1_rb @ W2r_rb + b2_rb) @ Wc1u)[None, :]

  V2m = W2l_r @ Wc1m
  W2rm = W2r_r @ Wc1m
  N1 = W1l_rb @ V2m
  N23 = W1r_rb @ V2m + W1l_r @ W2rm
  v2 = (b1_rb @ V2m)[None, :]
  N4 = W1r_r @ W2rm
  c_m = ((b1_r @ W2r_r + b2_r) @ Wc1m)[None, :]

  pad32 = lambda w: jnp.zeros((W32, 64), _f32).at[:w.shape[0]].set(w)
  M1p, M23p, V4p = pad32(M1), pad32(M23), pad32(V4)
  N1p, N23p, N4p = pad32(N1), pad32(N23), pad32(N4)

  Sm, Su = _seg_sum_sc(src2, dst2, xu_ext, xm_ext, zrows)

  aggU = _agg_table_tc(Su, 18)
  aggM = _agg_table_tc(Sm, 24)

  Pm, Qu = _seg_sum_sc(src2, dst2, aggU, aggM, zrows)

  A_u = _a_table_tc(Qu, Su, xu_ext, M1p, M23p, V4p, v1, c_u, 18)
  A_m = _a_table_tc(Pm, Sm, xm_ext, N1p, N23p, N4p, v2, c_m, 24)

  Gu, Gm = _gather_sc(ls1, ld1, A_u, A_m)

  out2 = _classifier_tc(Gu, Gm, bc1[None, :], Wc2, bc2[None, :],
                        Wc3[:, 0][None, :], bc3[None, :])
  return out2.reshape(L_PAD)[:L]


# speedup vs baseline: 7.4606x; 1.0949x over previous
import functools

import jax
import jax.numpy as jnp
from jax import lax
from jax.experimental import pallas as pl
from jax.experimental.pallas import tpu as pltpu
from jax.experimental.pallas import tpu_sc as plsc

NU = 50000
NM = 10000
E = 600000
L = 100000
H = 128

NC = 2
NS = 16
NW = NC * NS

W32 = 32
NU_A = 50176
NM_A = 10240
ZU = NU_A // NS
ZM = NM_A // NS

TPT = 304
GR = 16
NGR = TPT // GR
NSLOT = 4
GDEPTH = 2
EC = NS * TPT
E_PAD = EC * 128

PTC = 25
KBC = 5
NBLKC = PTC // KBC
L_PAD = NW * PTC * 128
LC = L_PAD // 128

BR = 512

_f32 = jnp.float32
_i32 = jnp.int32


def _seg_sum_sc(edges3, tab_u, tab_m, zrows):
  def body(edges_hbm, tabu_hbm, tabm_hbm, zero_hbm,
           accm_out, accu_out,
           idx, rows, acc,
           gsem0, gsem1, gsem2, gsem3, ssem0, ssem1, ssem2, ssem3):
    c = lax.axis_index("c")
    s = lax.axis_index("s")
    gsems = [gsem0, gsem1, gsem2, gsem3]
    ssems = [ssem0, ssem1, ssem2, ssem3]

    def run(gsel, ssel, tab_hbm, nrows, out_hbm):
      zr = nrows // NS
      pltpu.sync_copy(zero_hbm.at[pl.ds(0, zr)], acc.at[pl.ds(s * zr, zr)])
      plsc.subcore_barrier()
      start = s * TPT

      def grp(g, carry):
        base = start + g * GR
        pltpu.sync_copy(edges_hbm.at[pl.ds(base, GR)], idx)
        gdesc = [None] * NSLOT
        sdesc = [None] * NSLOT
        for h in range(GR + GDEPTH):
          slot = h % NSLOT
          if h < GR:
            if sdesc[slot] is not None:
              sdesc[slot].wait()
            gdesc[slot] = pltpu.async_copy(
                tab_hbm.at[idx.at[h, gsel]],
                rows.at[slot], gsems[slot])
          hp = h - GDEPTH
          if hp >= 0:
            p = hp % NSLOT
            gdesc[p].wait()
            sdesc[p] = pltpu.async_copy(
                rows.at[p], acc.at[idx.at[hp, ssel]], ssems[p], add=True)
        for d in sdesc:
          d.wait()
        return carry

      lax.fori_loop(0, NGR, grp, 0)
      plsc.subcore_barrier()
      pltpu.sync_copy(acc.at[pl.ds(s * zr, zr)], out_hbm.at[pl.ds(s * zr, zr)])

    @pl.when(c == 0)
    def _():
      run(0, 1, tabu_hbm, NM_A, accm_out)

    @pl.when(c == 1)
    def _():
      run(1, 0, tabm_hbm, NU_A, accu_out)

  mesh = plsc.VectorSubcoreMesh(core_axis_name="c", subcore_axis_name="s")
  f = pl.kernel(
      body,
      out_type=[
          jax.ShapeDtypeStruct((NM_A, W32), _f32),
          jax.ShapeDtypeStruct((NU_A, W32), _f32),
      ],
      mesh=mesh,
      scratch_types=[
          pltpu.VMEM((GR, 2, 128), _i32),
          pltpu.VMEM((NSLOT, 128, W32), _f32),
          pltpu.VMEM_SHARED((NU_A, W32), _f32),
      ] + [pltpu.SemaphoreType.DMA] * (2 * NSLOT),
      compiler_params=pltpu.CompilerParams(use_tc_tiling_on_sc=False),
  )
  return f(edges3, tab_u, tab_m, zrows)


def _gather_sc(lsrc, ldst, tab_u, tab_m):
  def body(ls_hbm, ld_hbm, tu_hbm, tm_hbm, gu_out, gm_out,
           idx_s, idx_d, rows_u, rows_m,
           gsem0, gsem1, osem0, osem1):
    c = lax.axis_index("c")
    s = lax.axis_index("s")
    start = (c * NS + s) * PTC * 128
    gsems = [gsem0, gsem1]
    osems = [osem0, osem1]
    pltpu.sync_copy(ls_hbm.at[pl.ds(start, PTC * 128)], idx_s)
    pltpu.sync_copy(ld_hbm.at[pl.ds(start, PTC * 128)], idx_d)
    gdesc = [None, None]
    odesc = [None, None]
    for h in range(PTC + 1):
      slot = h % 2
      if h < PTC:
        if odesc[slot] is not None:
          for d in odesc[slot]:
            d.wait()
        gdesc[slot] = [
            pltpu.async_copy(
                tu_hbm.at[idx_s.at[pl.ds(h * 128, 128)]],
                rows_u.at[slot], gsems[slot]),
            pltpu.async_copy(
                tm_hbm.at[idx_d.at[pl.ds(h * 128, 128)]],
                rows_m.at[slot], gsems[slot]),
        ]
      hp = h - 1
      if hp >= 0:
        p = hp % 2
        for d in gdesc[p]:
          d.wait()
        odesc[p] = [
            pltpu.async_copy(rows_u.at[p],
                             gu_out.at[pl.ds(start + hp * 128, 128)],
                             osems[p]),
            pltpu.async_copy(rows_m.at[p],
                             gm_out.at[pl.ds(start + hp * 128, 128)],
                             osems[p]),
        ]
    for ds_ in odesc:
      for d in ds_:
        d.wait()

  mesh = plsc.VectorSubcoreMesh(core_axis_name="c", subcore_axis_name="s")
  f = pl.kernel(
      body,
      out_type=[
          jax.ShapeDtypeStruct((L_PAD, 64), _f32),
          jax.ShapeDtypeStruct((L_PAD, 64), _f32),
      ],
      mesh=mesh,
      scratch_types=[
          pltpu.VMEM((PTC * 128,), _i32),
          pltpu.VMEM((PTC * 128,), _i32),
          pltpu.VMEM((2, 128, 64), _f32),
          pltpu.VMEM((2, 128, 64), _f32),
          pltpu.SemaphoreType.DMA,
          pltpu.SemaphoreType.DMA,
          pltpu.SemaphoreType.DMA,
          pltpu.SemaphoreType.DMA,
      ],
      compiler_params=pltpu.CompilerParams(use_tc_tiling_on_sc=False),
  )
  return f(lsrc, ldst, tab_u, tab_m)


def _agg_table_tc(s_tab, feat):
  n = s_tab.shape[0]

  def body(sp_ref, out_ref):
    S = sp_ref[...]
    m = jnp.maximum(S[:, feat], 1.0)
    col = lax.broadcasted_iota(_i32, (BR, W32), 1)
    out_ref[...] = jnp.where(col < feat, S / m[:, None], 0.0)

  return pl.pallas_call(
      body,
      grid=(n // BR,),
      in_specs=[pl.BlockSpec((BR, W32), lambda i: (i, 0))],
      out_specs=pl.BlockSpec((BR, W32), lambda i: (i, 0)),
      out_shape=jax.ShapeDtypeStruct((n, W32), _f32),
  )(s_tab)


def _a_table_tc(q_tab, s_tab, x_tab, wq, ws, wx, vb, cb, feat):
  n = x_tab.shape[0]

  def body(q_ref, s_ref, x_ref, wq_ref, ws_ref, wx_ref, vb_ref, cb_ref,
           out_ref):
    Q = q_ref[...]
    S = s_ref[...]
    deg = S[:, feat]
    m = jnp.maximum(deg, 1.0)
    br = (jnp.dot(Q, wq_ref[...], preferred_element_type=_f32)
          + jnp.dot(S, ws_ref[...], preferred_element_type=_f32)
          + deg[:, None] * vb_ref[...])
    out_ref[...] = (br / m[:, None]
                    + jnp.dot(x_ref[...], wx_ref[...],
                              preferred_element_type=_f32)
                    + cb_ref[...])

  full = lambda i: (0, 0)
  return pl.pallas_call(
      body,
      grid=(n // BR,),
      in_specs=[
          pl.BlockSpec((BR, W32), lambda i: (i, 0)),
          pl.BlockSpec((BR, W32), lambda i: (i, 0)),
          pl.BlockSpec((BR, W32), lambda i: (i, 0)),
          pl.BlockSpec((W32, 64), full),
          pl.BlockSpec((W32, 64), full),
          pl.BlockSpec((W32, 64), full),
          pl.BlockSpec((1, 64), full),
          pl.BlockSpec((1, 64), full),
      ],
      out_specs=pl.BlockSpec((BR, 64), lambda i: (i, 0)),
      out_shape=jax.ShapeDtypeStruct((n, 64), _f32),
  )(q_tab, s_tab, x_tab, wq, ws, wx, vb, cb)


def _classifier_tc(gu, gm, b1, w2, b2, w3row, b3):
  def body(gu_ref, gm_ref, b1_ref, w2_ref, b2_ref, w3_ref, b3_ref, out_ref):
    h = jnp.maximum(gu_ref[...] + gm_ref[...] + b1_ref[...], 0.0)
    h2 = jnp.maximum(
        jnp.dot(h, w2_ref[...], preferred_element_type=_f32) + b2_ref[...],
        0.0)
    z = jnp.sum(h2 * w3_ref[...], axis=1) + b3_ref[0, 0]
    out_ref[...] = (5.0 / (1.0 + jnp.exp(-z)))[:, None]

  full = lambda i: (0, 0)
  return pl.pallas_call(
      body,
      grid=(L_PAD // BR,),
      in_specs=[
          pl.BlockSpec((BR, 64), lambda i: (i, 0)),
          pl.BlockSpec((BR, 64), lambda i: (i, 0)),
          pl.BlockSpec((1, 64), full),
          pl.BlockSpec((64, 16), full),
          pl.BlockSpec((1, 16), full),
          pl.BlockSpec((1, 16), full),
          pl.BlockSpec((1, 1), full),
      ],
      out_specs=pl.BlockSpec((BR, 1), lambda i: (i, 0)),
      out_shape=jax.ShapeDtypeStruct((L_PAD, 1), _f32),
  )(gu, gm, b1, w2, b2, w3row, b3)


def kernel(x_user, x_movie, edge_src, edge_dst, label_src, label_dst,
           W1l_r, b1_r, W1r_r, W1l_rb, b1_rb, W1r_rb,
           W2l_r, b2_r, W2r_r, W2l_rb, b2_rb, W2r_rb,
           Wc1, bc1, Wc2, bc2, Wc3, bc3):
  es = edge_src.astype(_i32)
  ed = edge_dst.astype(_i32)
  src2 = jnp.concatenate(
      [es, jnp.full((E_PAD - E,), NU, _i32)]).reshape(EC, 128)
  dst2 = jnp.concatenate(
      [ed, jnp.full((E_PAD - E,), NM, _i32)]).reshape(EC, 128)
  edges3 = jnp.stack([src2, dst2], axis=1)
  ls1 = jnp.concatenate(
      [label_src.astype(_i32), jnp.zeros((L_PAD - L,), _i32)])
  ld1 = jnp.concatenate(
      [label_dst.astype(_i32), jnp.zeros((L_PAD - L,), _i32)])

  xu_ext = jnp.zeros((NU_A, W32), _f32)
  xu_ext = xu_ext.at[:NU, :24].set(x_user).at[:NU, 24].set(1.0)
  xm_ext = jnp.zeros((NM_A, W32), _f32)
  xm_ext = xm_ext.at[:NM, :18].set(x_movie).at[:NM, 18].set(1.0)
  zrows = jnp.zeros((ZU, W32), _f32)

  Wc1u, Wc1m = Wc1[:H], Wc1[H:]
  V1 = W2l_rb @ Wc1u
  W2ru = W2r_rb @ Wc1u
  M1 = W1l_r @ V1
  M23 = W1r_r @ V1 + W1l_rb @ W2ru
  v1 = (b1_r @ V1)[None, :]
  V4 = W1r_rb @ W2ru
  c_u = ((b1_rb @ W2r_rb + b2_rb) @ Wc1u)[None, :]

  V2m = W2l_r @ Wc1m
  W2rm = W2r_r @ Wc1m
  N1 = W1l_rb @ V2m
  N23 = W1r_rb @ V2m + W1l_r @ W2rm
  v2 = (b1_rb @ V2m)[None, :]
  N4 = W1r_r @ W2rm
  c_m = ((b1_r @ W2r_r + b2_r) @ Wc1m)[None, :]

  pad32 = lambda w: jnp.zeros((W32, 64), _f32).at[:w.shape[0]].set(w)
  M1p, M23p, V4p = pad32(M1), pad32(M23), pad32(V4)
  N1p, N23p, N4p = pad32(N1), pad32(N23), pad32(N4)

  Sm, Su = _seg_sum_sc(edges3, xu_ext, xm_ext, zrows)

  aggU = _agg_table_tc(Su, 18)
  aggM = _agg_table_tc(Sm, 24)

  Pm, Qu = _seg_sum_sc(edges3, aggU, aggM, zrows)

  A_u = _a_table_tc(Qu, Su, xu_ext, M1p, M23p, V4p, v1, c_u, 18)
  A_m = _a_table_tc(Pm, Sm, xm_ext, N1p, N23p, N4p, v2, c_m, 24)

  Gu, Gm = _gather_sc(ls1, ld1, A_u, A_m)

  out2 = _classifier_tc(Gu, Gm, bc1[None, :], Wc2, bc2[None, :],
                        Wc3[:, 0][None, :], bc3[None, :])
  return out2.reshape(L_PAD)[:L]


# speedup vs baseline: 7.5295x; 1.0092x over previous
import functools

import jax
import jax.numpy as jnp
from jax import lax
from jax.experimental import pallas as pl
from jax.experimental.pallas import tpu as pltpu
from jax.experimental.pallas import tpu_sc as plsc

NU = 50000
NM = 10000
E = 600000
L = 100000
H = 128

NC = 2
NS = 16
NW = NC * NS

W32 = 32
NU_A = 50176
NM_A = 10240
ZU = NU_A // NS
ZM = NM_A // NS

TPT = 304
GR = 16
NGR = TPT // GR
NSLOT = 6
GDEPTH = 4
EC = NS * TPT
E_PAD = EC * 128

PTC = 25
KBC = 5
NBLKC = PTC // KBC
L_PAD = NW * PTC * 128
LC = L_PAD // 128

BR = 512

_f32 = jnp.float32
_i32 = jnp.int32


def _seg_sum_sc(edges3, tab_u, tab_m, zrows):
  def body(edges_hbm, tabu_hbm, tabm_hbm, zero_hbm,
           accm_out, accu_out,
           idx, rows, acc, *sems):
    c = lax.axis_index("c")
    s = lax.axis_index("s")
    gsems = list(sems[:NSLOT])
    ssems = list(sems[NSLOT:])

    def run(gsel, ssel, tab_hbm, nrows, out_hbm):
      zr = nrows // NS
      pltpu.sync_copy(zero_hbm.at[pl.ds(0, zr)], acc.at[pl.ds(s * zr, zr)])
      plsc.subcore_barrier()
      start = s * TPT

      def grp(g, carry):
        base = start + g * GR
        pltpu.sync_copy(edges_hbm.at[pl.ds(base, GR)], idx)
        gdesc = [None] * NSLOT
        sdesc = [None] * NSLOT
        for h in range(GR + GDEPTH):
          slot = h % NSLOT
          if h < GR:
            if sdesc[slot] is not None:
              sdesc[slot].wait()
            gdesc[slot] = pltpu.async_copy(
                tab_hbm.at[idx.at[h, gsel]],
                rows.at[slot], gsems[slot])
          hp = h - GDEPTH
          if hp >= 0:
            p = hp % NSLOT
            gdesc[p].wait()
            sdesc[p] = pltpu.async_copy(
                rows.at[p], acc.at[idx.at[hp, ssel]], ssems[p], add=True)
        for d in sdesc:
          d.wait()
        return carry

      lax.fori_loop(0, NGR, grp, 0)
      plsc.subcore_barrier()
      pltpu.sync_copy(acc.at[pl.ds(s * zr, zr)], out_hbm.at[pl.ds(s * zr, zr)])

    @pl.when(c == 0)
    def _():
      run(0, 1, tabu_hbm, NM_A, accm_out)

    @pl.when(c == 1)
    def _():
      run(1, 0, tabm_hbm, NU_A, accu_out)

  mesh = plsc.VectorSubcoreMesh(core_axis_name="c", subcore_axis_name="s")
  f = pl.kernel(
      body,
      out_type=[
          jax.ShapeDtypeStruct((NM_A, W32), _f32),
          jax.ShapeDtypeStruct((NU_A, W32), _f32),
      ],
      mesh=mesh,
      scratch_types=[
          pltpu.VMEM((GR, 2, 128), _i32),
          pltpu.VMEM((NSLOT, 128, W32), _f32),
          pltpu.VMEM_SHARED((NU_A, W32), _f32),
      ] + [pltpu.SemaphoreType.DMA] * (2 * NSLOT),
      compiler_params=pltpu.CompilerParams(use_tc_tiling_on_sc=False),
  )
  return f(edges3, tab_u, tab_m, zrows)


def _gather_sc(lsrc, ldst, tab_u, tab_m):
  def body(ls_hbm, ld_hbm, tu_hbm, tm_hbm, gu_out, gm_out,
           idx_s, idx_d, rows_u, rows_m,
           gsem0, gsem1, osem0, osem1):
    c = lax.axis_index("c")
    s = lax.axis_index("s")
    start = (c * NS + s) * PTC * 128
    gsems = [gsem0, gsem1]
    osems = [osem0, osem1]
    pltpu.sync_copy(ls_hbm.at[pl.ds(start, PTC * 128)], idx_s)
    pltpu.sync_copy(ld_hbm.at[pl.ds(start, PTC * 128)], idx_d)
    gdesc = [None, None]
    odesc = [None, None]
    for h in range(PTC + 1):
      slot = h % 2
      if h < PTC:
        if odesc[slot] is not None:
          for d in odesc[slot]:
            d.wait()
        gdesc[slot] = [
            pltpu.async_copy(
                tu_hbm.at[idx_s.at[pl.ds(h * 128, 128)]],
                rows_u.at[slot], gsems[slot]),
            pltpu.async_copy(
                tm_hbm.at[idx_d.at[pl.ds(h * 128, 128)]],
                rows_m.at[slot], gsems[slot]),
        ]
      hp = h - 1
      if hp >= 0:
        p = hp % 2
        for d in gdesc[p]:
          d.wait()
        odesc[p] = [
            pltpu.async_copy(rows_u.at[p],
                             gu_out.at[pl.ds(start + hp * 128, 128)],
                             osems[p]),
            pltpu.async_copy(rows_m.at[p],
                             gm_out.at[pl.ds(start + hp * 128, 128)],
                             osems[p]),
        ]
    for ds_ in odesc:
      for d in ds_:
        d.wait()

  mesh = plsc.VectorSubcoreMesh(core_axis_name="c", subcore_axis_name="s")
  f = pl.kernel(
      body,
      out_type=[
          jax.ShapeDtypeStruct((L_PAD, 64), _f32),
          jax.ShapeDtypeStruct((L_PAD, 64), _f32),
      ],
      mesh=mesh,
      scratch_types=[
          pltpu.VMEM((PTC * 128,), _i32),
          pltpu.VMEM((PTC * 128,), _i32),
          pltpu.VMEM((2, 128, 64), _f32),
          pltpu.VMEM((2, 128, 64), _f32),
          pltpu.SemaphoreType.DMA,
          pltpu.SemaphoreType.DMA,
          pltpu.SemaphoreType.DMA,
          pltpu.SemaphoreType.DMA,
      ],
      compiler_params=pltpu.CompilerParams(use_tc_tiling_on_sc=False),
  )
  return f(lsrc, ldst, tab_u, tab_m)


def _agg_table_tc(s_tab, feat):
  n = s_tab.shape[0]

  def body(sp_ref, out_ref):
    S = sp_ref[...]
    m = jnp.maximum(S[:, feat], 1.0)
    col = lax.broadcasted_iota(_i32, (BR, W32), 1)
    out_ref[...] = jnp.where(col < feat, S / m[:, None], 0.0)

  return pl.pallas_call(
      body,
      grid=(n // BR,),
      in_specs=[pl.BlockSpec((BR, W32), lambda i: (i, 0))],
      out_specs=pl.BlockSpec((BR, W32), lambda i: (i, 0)),
      out_shape=jax.ShapeDtypeStruct((n, W32), _f32),
  )(s_tab)


def _a_table_tc(q_tab, s_tab, x_tab, wq, ws, wx, vb, cb, feat):
  n = x_tab.shape[0]

  def body(q_ref, s_ref, x_ref, wq_ref, ws_ref, wx_ref, vb_ref, cb_ref,
           out_ref):
    Q = q_ref[...]
    S = s_ref[...]
    deg = S[:, feat]
    m = jnp.maximum(deg, 1.0)
    br = (jnp.dot(Q, wq_ref[...], preferred_element_type=_f32)
          + jnp.dot(S, ws_ref[...], preferred_element_type=_f32)
          + deg[:, None] * vb_ref[...])
    out_ref[...] = (br / m[:, None]
                    + jnp.dot(x_ref[...], wx_ref[...],
                              preferred_element_type=_f32)
                    + cb_ref[...])

  full = lambda i: (0, 0)
  return pl.pallas_call(
      body,
      grid=(n // BR,),
      in_specs=[
          pl.BlockSpec((BR, W32), lambda i: (i, 0)),
          pl.BlockSpec((BR, W32), lambda i: (i, 0)),
          pl.BlockSpec((BR, W32), lambda i: (i, 0)),
          pl.BlockSpec((W32, 64), full),
          pl.BlockSpec((W32, 64), full),
          pl.BlockSpec((W32, 64), full),
          pl.BlockSpec((1, 64), full),
          pl.BlockSpec((1, 64), full),
      ],
      out_specs=pl.BlockSpec((BR, 64), lambda i: (i, 0)),
      out_shape=jax.ShapeDtypeStruct((n, 64), _f32),
  )(q_tab, s_tab, x_tab, wq, ws, wx, vb, cb)


def _classifier_tc(gu, gm, b1, w2, b2, w3row, b3):
  def body(gu_ref, gm_ref, b1_ref, w2_ref, b2_ref, w3_ref, b3_ref, out_ref):
    h = jnp.maximum(gu_ref[...] + gm_ref[...] + b1_ref[...], 0.0)
    h2 = jnp.maximum(
        jnp.dot(h, w2_ref[...], preferred_element_type=_f32) + b2_ref[...],
        0.0)
    z = jnp.sum(h2 * w3_ref[...], axis=1) + b3_ref[0, 0]
    out_ref[...] = (5.0 / (1.0 + jnp.exp(-z)))[:, None]

  full = lambda i: (0, 0)
  return pl.pallas_call(
      body,
      grid=(L_PAD // BR,),
      in_specs=[
          pl.BlockSpec((BR, 64), lambda i: (i, 0)),
          pl.BlockSpec((BR, 64), lambda i: (i, 0)),
          pl.BlockSpec((1, 64), full),
          pl.BlockSpec((64, 16), full),
          pl.BlockSpec((1, 16), full),
          pl.BlockSpec((1, 16), full),
          pl.BlockSpec((1, 1), full),
      ],
      out_specs=pl.BlockSpec((BR, 1), lambda i: (i, 0)),
      out_shape=jax.ShapeDtypeStruct((L_PAD, 1), _f32),
  )(gu, gm, b1, w2, b2, w3row, b3)


def kernel(x_user, x_movie, edge_src, edge_dst, label_src, label_dst,
           W1l_r, b1_r, W1r_r, W1l_rb, b1_rb, W1r_rb,
           W2l_r, b2_r, W2r_r, W2l_rb, b2_rb, W2r_rb,
           Wc1, bc1, Wc2, bc2, Wc3, bc3):
  es = edge_src.astype(_i32)
  ed = edge_dst.astype(_i32)
  src2 = jnp.concatenate(
      [es, jnp.full((E_PAD - E,), NU, _i32)]).reshape(EC, 128)
  dst2 = jnp.concatenate(
      [ed, jnp.full((E_PAD - E,), NM, _i32)]).reshape(EC, 128)
  edges3 = jnp.stack([src2, dst2], axis=1)
  ls1 = jnp.concatenate(
      [label_src.astype(_i32), jnp.zeros((L_PAD - L,), _i32)])
  ld1 = jnp.concatenate(
      [label_dst.astype(_i32), jnp.zeros((L_PAD - L,), _i32)])

  xu_ext = jnp.zeros((NU_A, W32), _f32)
  xu_ext = xu_ext.at[:NU, :24].set(x_user).at[:NU, 24].set(1.0)
  xm_ext = jnp.zeros((NM_A, W32), _f32)
  xm_ext = xm_ext.at[:NM, :18].set(x_movie).at[:NM, 18].set(1.0)
  zrows = jnp.zeros((ZU, W32), _f32)

  Wc1u, Wc1m = Wc1[:H], Wc1[H:]
  V1 = W2l_rb @ Wc1u
  W2ru = W2r_rb @ Wc1u
  M1 = W1l_r @ V1
  M23 = W1r_r @ V1 + W1l_rb @ W2ru
  v1 = (b1_r @ V1)[None, :]
  V4 = W1r_rb @ W2ru
  c_u = ((b1_rb @ W2r_rb + b2_rb) @ Wc1u)[None, :]

  V2m = W2l_r @ Wc1m
  W2rm = W2r_r @ Wc1m
  N1 = W1l_rb @ V2m
  N23 = W1r_rb @ V2m + W1l_r @ W2rm
  v2 = (b1_rb @ V2m)[None, :]
  N4 = W1r_r @ W2rm
  c_m = ((b1_r @ W2r_r + b2_r) @ Wc1m)[None, :]

  pad32 = lambda w: jnp.zeros((W32, 64), _f32).at[:w.shape[0]].set(w)
  M1p, M23p, V4p = pad32(M1), pad32(M23), pad32(V4)
  N1p, N23p, N4p = pad32(N1), pad32(N23), pad32(N4)

  Sm, Su = _seg_sum_sc(edges3, xu_ext, xm_ext, zrows)

  aggU = _agg_table_tc(Su, 18)
  aggM = _agg_table_tc(Sm, 24)

  Pm, Qu = _seg_sum_sc(edges3, aggU, aggM, zrows)

  A_u = _a_table_tc(Qu, Su, xu_ext, M1p, M23p, V4p, v1, c_u, 18)
  A_m = _a_table_tc(Pm, Sm, xm_ext, N1p, N23p, N4p, v2, c_m, 24)

  Gu, Gm = _gather_sc(ls1, ld1, A_u, A_m)

  out2 = _classifier_tc(Gu, Gm, bc1[None, :], Wc2, bc2[None, :],
                        Wc3[:, 0][None, :], bc3[None, :])
  return out2.reshape(L_PAD)[:L]


# speedup vs baseline: 9.6506x; 1.2817x over previous
import functools

import jax
import jax.numpy as jnp
from jax import lax
from jax.experimental import pallas as pl
from jax.experimental.pallas import tpu as pltpu
from jax.experimental.pallas import tpu_sc as plsc

NU = 50000
NM = 10000
E = 600000
L = 100000
H = 128

NC = 2
NS = 16
NW = NC * NS

W32 = 32
NU_A = 50176
NM_A = 10240
ZU = NU_A // NS
ZM = NM_A // NS

TPT = 304
GR = 16
NGR = TPT // GR
NSLOT = 6
GDEPTH = 4
EC = NS * TPT
E_PAD = EC * 128

PTC = 25
KBC = 5
NBLKC = PTC // KBC
L_PAD = NW * PTC * 128
LC = L_PAD // 128

BR = 512

_f32 = jnp.float32
_i32 = jnp.int32


def _seg_sum_sc(edges3, tab_u, tab_m, zrows):
  def body(edges_hbm, tabu_hbm, tabm_hbm, zero_hbm,
           accm_out, accu_out,
           idx, rows, acc, *sems):
    c = lax.axis_index("c")
    s = lax.axis_index("s")
    gsems = list(sems[:NSLOT])
    ssems = list(sems[NSLOT:])

    def run(gsel, ssel, tab_hbm, nrows, out_hbm):
      zr = nrows // NS
      pltpu.sync_copy(zero_hbm.at[pl.ds(0, zr)], acc.at[pl.ds(s * zr, zr)])
      plsc.subcore_barrier()

      def grp(g, carry):
        base = (s * NGR + g) * 2 * GR
        pltpu.sync_copy(edges_hbm.at[pl.ds(base, 2 * GR)], idx)
        gdesc = [None] * NSLOT
        sdesc = [None] * NSLOT
        for h in range(GR + GDEPTH):
          slot = h % NSLOT
          if h < GR:
            if sdesc[slot] is not None:
              sdesc[slot].wait()
            gdesc[slot] = pltpu.async_copy(
                tab_hbm.at[idx.at[gsel * GR + h]],
                rows.at[slot], gsems[slot])
          hp = h - GDEPTH
          if hp >= 0:
            p = hp % NSLOT
            gdesc[p].wait()
            sdesc[p] = pltpu.async_copy(
                rows.at[p], acc.at[idx.at[ssel * GR + hp]], ssems[p],
                add=True)
        for d in sdesc:
          d.wait()
        return carry

      lax.fori_loop(0, NGR, grp, 0)
      plsc.subcore_barrier()
      pltpu.sync_copy(acc.at[pl.ds(s * zr, zr)], out_hbm.at[pl.ds(s * zr, zr)])

    @pl.when(c == 0)
    def _():
      run(0, 1, tabu_hbm, NM_A, accm_out)

    @pl.when(c == 1)
    def _():
      run(1, 0, tabm_hbm, NU_A, accu_out)

  mesh = plsc.VectorSubcoreMesh(core_axis_name="c", subcore_axis_name="s")
  f = pl.kernel(
      body,
      out_type=[
          jax.ShapeDtypeStruct((NM_A, W32), _f32),
          jax.ShapeDtypeStruct((NU_A, W32), _f32),
      ],
      mesh=mesh,
      scratch_types=[
          pltpu.VMEM((2 * GR, 128), _i32),
          pltpu.VMEM((NSLOT, 128, W32), _f32),
          pltpu.VMEM_SHARED((NU_A, W32), _f32),
      ] + [pltpu.SemaphoreType.DMA] * (2 * NSLOT),
      compiler_params=pltpu.CompilerParams(use_tc_tiling_on_sc=False),
  )
  return f(edges3, tab_u, tab_m, zrows)


def _gather_sc(lsrc, ldst, tab_u, tab_m):
  def body(ls_hbm, ld_hbm, tu_hbm, tm_hbm, h0_out,
           idx_s, idx_d, rows_u, rows_m,
           gsem0, gsem1, osem0, osem1):
    c = lax.axis_index("c")
    s = lax.axis_index("s")
    start = (c * NS + s) * PTC * 128
    gsems = [gsem0, gsem1]
    osems = [osem0, osem1]
    pltpu.sync_copy(ls_hbm.at[pl.ds(start, PTC * 128)], idx_s)
    pltpu.sync_copy(ld_hbm.at[pl.ds(start, PTC * 128)], idx_d)

    def fire(h, slot):
      return [
          pltpu.async_copy(
              tu_hbm.at[idx_s.at[pl.ds(h * 128, 128)]],
              rows_u.at[slot], gsems[slot]),
          pltpu.async_copy(
              tm_hbm.at[idx_d.at[pl.ds(h * 128, 128)]],
              rows_m.at[slot], gsems[slot]),
      ]

    def addout(h, slot, gd):
      for d in gd:
        d.wait()
      for r in range(128):
        for k in range(4):
          rows_u[slot, r, pl.ds(k * 16, 16)] = (
              rows_u[slot, r, pl.ds(k * 16, 16)]
              + rows_m[slot, r, pl.ds(k * 16, 16)])
      return pltpu.async_copy(rows_u.at[slot],
                              h0_out.at[pl.ds(start + h * 128, 128)],
                              osems[slot])

    def pair(i, carry):
      h0i = i * 2
      gd0 = fire(h0i, 0)
      gd1 = fire(h0i + 1, 1)
      od0 = addout(h0i, 0, gd0)
      od1 = addout(h0i + 1, 1, gd1)
      od0.wait()
      od1.wait()
      return carry

    lax.fori_loop(0, PTC // 2, pair, 0)
    h = PTC - 1
    gd = fire(h, 0)
    addout(h, 0, gd).wait()

  mesh = plsc.VectorSubcoreMesh(core_axis_name="c", subcore_axis_name="s")
  f = pl.kernel(
      body,
      out_type=[jax.ShapeDtypeStruct((L_PAD, 64), _f32)],
      mesh=mesh,
      scratch_types=[
          pltpu.VMEM((PTC * 128,), _i32),
          pltpu.VMEM((PTC * 128,), _i32),
          pltpu.VMEM((2, 128, 64), _f32),
          pltpu.VMEM((2, 128, 64), _f32),
          pltpu.SemaphoreType.DMA,
          pltpu.SemaphoreType.DMA,
          pltpu.SemaphoreType.DMA,
          pltpu.SemaphoreType.DMA,
      ],
      compiler_params=pltpu.CompilerParams(use_tc_tiling_on_sc=False),
  )
  (h0,) = f(lsrc, ldst, tab_u, tab_m)
  return h0


def _agg_table_tc(s_tab, feat):
  n = s_tab.shape[0]
  s4 = s_tab.reshape(n // 4, 128)
  esel = jnp.zeros((128, 4), _f32).at[W32 * jnp.arange(4) + feat,
                                      jnp.arange(4)].set(1.0)
  r = jnp.arange(128)
  spread = jnp.where((r % W32 < feat)[None, :]
                     & (r[None, :] // W32 == jnp.arange(4)[:, None]),
                     1.0, 0.0)

  def body(s_ref, e_ref, g_ref, out_ref):
    S = s_ref[...]
    deg4 = jnp.dot(S, e_ref[...], preferred_element_type=_f32)
    rec = 1.0 / jnp.maximum(deg4, 1.0)
    out_ref[...] = S * jnp.dot(rec, g_ref[...], preferred_element_type=_f32)

  full = lambda i: (0, 0)
  out4 = pl.pallas_call(
      body,
      grid=(n // 4 // 128,),
      in_specs=[
          pl.BlockSpec((128, 128), lambda i: (i, 0)),
          pl.BlockSpec((128, 4), full),
          pl.BlockSpec((4, 128), full),
      ],
      out_specs=pl.BlockSpec((128, 128), lambda i: (i, 0)),
      out_shape=jax.ShapeDtypeStruct((n // 4, 128), _f32),
  )(s4, esel, spread)
  return out4.reshape(n, W32)


def _a_table_tc(q_tab, s_tab, x_tab, wq, ws, wx, vb, cb, feat):
  n = x_tab.shape[0]
  q4 = q_tab.reshape(n // 4, 128)
  s4 = s_tab.reshape(n // 4, 128)
  x4 = x_tab.reshape(n // 4, 128)
  eye4 = jnp.eye(4, dtype=_f32)
  wqb = jnp.kron(eye4, wq)
  wsb = jnp.kron(eye4, ws)
  wxb = jnp.kron(eye4, wx)
  vbb = jnp.kron(eye4, vb)
  gdb = jnp.kron(eye4, jnp.ones((1, 64), _f32))
  cbb = jnp.tile(cb, (1, 4))
  esel = jnp.zeros((128, 4), _f32).at[W32 * jnp.arange(4) + feat,
                                      jnp.arange(4)].set(1.0)

  def body(q_ref, s_ref, x_ref, wq_ref, ws_ref, wx_ref, vb_ref, gd_ref,
           cb_ref, e_ref, out_ref):
    Q = q_ref[...]
    S = s_ref[...]
    deg4 = jnp.dot(S, e_ref[...], preferred_element_type=_f32)
    rec = 1.0 / jnp.maximum(deg4, 1.0)
    br = (jnp.dot(Q, wq_ref[...], preferred_element_type=_f32)
          + jnp.dot(S, ws_ref[...], preferred_element_type=_f32)
          + jnp.dot(deg4, vb_ref[...], preferred_element_type=_f32))
    a4 = (br * jnp.dot(rec, gd_ref[...], preferred_element_type=_f32)
          + jnp.dot(x_ref[...], wx_ref[...], preferred_element_type=_f32)
          + cb_ref[...])
    out_ref[...] = a4.reshape(256, 128)

  full = lambda i: (0, 0)
  a2 = pl.pallas_call(
      body,
      grid=(n // 4 // 128,),
      in_specs=[
          pl.BlockSpec((128, 128), lambda i: (i, 0)),
          pl.BlockSpec((128, 128), lambda i: (i, 0)),
          pl.BlockSpec((128, 128), lambda i: (i, 0)),
          pl.BlockSpec((128, 256), full),
          pl.BlockSpec((128, 256), full),
          pl.BlockSpec((128, 256), full),
          pl.BlockSpec((4, 256), full),
          pl.BlockSpec((4, 256), full),
          pl.BlockSpec((1, 256), full),
          pl.BlockSpec((128, 4), full),
      ],
      out_specs=pl.BlockSpec((256, 128), lambda i: (i, 0)),
      out_shape=jax.ShapeDtypeStruct((n // 2, 128), _f32),
  )(q4, s4, x4, wqb, wsb, wxb, vbb, gdb, cbb, esel)
  return a2.reshape(n, 64)


def _classifier_tc(h0, bc1, Wc2, bc2, wc3, bc3):
  h2d = h0.reshape(L_PAD // 2, 128)
  b1d = jnp.tile(bc1[None, :], (1, 2))
  w2b = jnp.kron(jnp.eye(2, dtype=_f32), Wc2)
  b2d = jnp.tile(bc2[None, :], (1, 2))
  w3d = jnp.tile(wc3[None, :], (1, 2))
  BRL = 512

  def body(h_ref, b1_ref, w2_ref, b2_ref, w3_ref, b3_ref, out_ref):
    h = jnp.maximum(h_ref[...] + b1_ref[...], 0.0)
    h2 = jnp.maximum(
        jnp.dot(h, w2_ref[...], preferred_element_type=_f32) + b2_ref[...],
        0.0)
    zz = h2 * w3_ref[...]
    z_e = jnp.sum(zz[:, :16], axis=1) + b3_ref[0, 0]
    z_o = jnp.sum(zz[:, 16:], axis=1) + b3_ref[0, 0]
    o_e = 5.0 / (1.0 + jnp.exp(-z_e))
    o_o = 5.0 / (1.0 + jnp.exp(-z_o))
    out_ref[...] = jnp.concatenate(
        [o_e.reshape(BRL // 64, 64), o_o.reshape(BRL // 64, 64)], axis=1)

  full = lambda i: (0, 0)
  return pl.pallas_call(
      body,
      grid=(L_PAD // 2 // BRL,),
      in_specs=[
          pl.BlockSpec((BRL, 128), lambda i: (i, 0)),
          pl.BlockSpec((1, 128), full),
          pl.BlockSpec((128, 32), full),
          pl.BlockSpec((1, 32), full),
          pl.BlockSpec((1, 32), full),
          pl.BlockSpec((1, 1), full),
      ],
      out_specs=pl.BlockSpec((BRL // 64, 128), lambda i: (i, 0)),
      out_shape=jax.ShapeDtypeStruct((L_PAD // 128, 128), _f32),
  )(h2d, b1d, w2b, b2d, w3d, bc3[None, :])


def kernel(x_user, x_movie, edge_src, edge_dst, label_src, label_dst,
           W1l_r, b1_r, W1r_r, W1l_rb, b1_rb, W1r_rb,
           W2l_r, b2_r, W2r_r, W2l_rb, b2_rb, W2r_rb,
           Wc1, bc1, Wc2, bc2, Wc3, bc3):
  es = edge_src.astype(_i32)
  ed = edge_dst.astype(_i32)
  TG = EC // GR
  srcg = jnp.concatenate(
      [es, jnp.full((E_PAD - E,), NU, _i32)]).reshape(TG, GR, 128)
  dstg = jnp.concatenate(
      [ed, jnp.full((E_PAD - E,), NM, _i32)]).reshape(TG, GR, 128)
  edges2 = jnp.concatenate([srcg, dstg], axis=1).reshape(TG * 2 * GR, 128)
  perm = lambda a: a.reshape(LC, 2, 64).swapaxes(1, 2).reshape(L_PAD)
  lsp = perm(jnp.concatenate(
      [label_src.astype(_i32), jnp.zeros((L_PAD - L,), _i32)]))
  ldp = perm(jnp.concatenate(
      [label_dst.astype(_i32), jnp.zeros((L_PAD - L,), _i32)]))

  def packed_table(x, feat, n_nodes, n_rows):
    fdim = x.shape[1]
    xg = jnp.pad(x.reshape(n_nodes // 4, 4 * fdim),
                 ((0, n_rows - n_nodes // 4), (0, 0)))
    r = jnp.arange(4 * fdim)
    spread = jnp.zeros((4 * fdim, 128), _f32).at[
        r, (r // fdim) * W32 + r % fdim].set(1.0)
    ones_cols = jnp.zeros((1, 128), _f32).at[
        0, W32 * jnp.arange(4) + feat].set(1.0)
    return (xg @ spread + ones_cols).reshape(n_rows * 4, W32)

  xu_ext = packed_table(x_user, 24, NU, NU_A // 4)
  xm_ext = packed_table(x_movie, 18, NM, NM_A // 4)
  zrows = jnp.zeros((ZU, W32), _f32)

  Wc1u, Wc1m = Wc1[:H], Wc1[H:]
  V1 = W2l_rb @ Wc1u
  W2ru = W2r_rb @ Wc1u
  M1 = W1l_r @ V1
  M23 = W1r_r @ V1 + W1l_rb @ W2ru
  v1 = (b1_r @ V1)[None, :]
  V4 = W1r_rb @ W2ru
  c_u = ((b1_rb @ W2r_rb + b2_rb) @ Wc1u)[None, :]

  V2m = W2l_r @ Wc1m
  W2rm = W2r_r @ Wc1m
  N1 = W1l_rb @ V2m
  N23 = W1r_rb @ V2m + W1l_r @ W2rm
  v2 = (b1_rb @ V2m)[None, :]
  N4 = W1r_r @ W2rm
  c_m = ((b1_r @ W2r_r + b2_r) @ Wc1m)[None, :]

  pad32 = lambda w: jnp.zeros((W32, 64), _f32).at[:w.shape[0]].set(w)
  M1p, M23p, V4p = pad32(M1), pad32(M23), pad32(V4)
  N1p, N23p, N4p = pad32(N1), pad32(N23), pad32(N4)

  Sm, Su = _seg_sum_sc(edges2, xu_ext, xm_ext, zrows)

  aggU = _agg_table_tc(Su, 18)
  aggM = _agg_table_tc(Sm, 24)

  Pm, Qu = _seg_sum_sc(edges2, aggU, aggM, zrows)

  A_u = _a_table_tc(Qu, Su, xu_ext, M1p, M23p, V4p, v1, c_u, 18)
  A_m = _a_table_tc(Pm, Sm, xm_ext, N1p, N23p, N4p, v2, c_m, 24)

  h0 = _gather_sc(lsp, ldp, A_u, A_m)

  out2 = _classifier_tc(h0, bc1, Wc2, bc2, Wc3[:, 0], bc3)
  return out2.reshape(L_PAD)[:L]
